# Initial kernel scaffold; baseline (speedup 1.0000x reference)
import functools

import jax
import jax.numpy as jnp
from jax import lax
from jax.experimental import pallas as pl
from jax.experimental.pallas import tpu as pltpu
from jax.experimental.pallas import tpu_sc as plsc

N = 10000
E = 320000
D = 128

NC = 2
NS = 16
NW = NC * NS
EPW = E // NW

PB = 125
NPB = EPW // PB
RPT = N // NS

HR = 640
HRPT = HR // NS


def _mesh():
    return plsc.VectorSubcoreMesh(core_axis_name="c", subcore_axis_name="s")


@functools.partial(
    pl.kernel,
    out_type=jax.ShapeDtypeStruct((NW, HRPT, 16), jnp.float32),
    scratch_types=[
        pltpu.VMEM((EPW // 16, 16), jnp.int32),
        pltpu.VMEM((HR // 128, 128), jnp.int32),
        pltpu.VMEM((HR, 16), jnp.float32),
        pltpu.VMEM_SHARED((HR, 16), jnp.float32),
    ],
    mesh=_mesh(),
)
def _hist(dst_ref, idid_ref, hp_ref, dstv, ididv, histv, spmem):
    cid = lax.axis_index("c")
    sid = lax.axis_index("s")
    w = cid * NS + sid
    pltpu.sync_copy(dst_ref.at[w], dstv)
    pltpu.sync_copy(idid_ref, ididv)
    z16 = jnp.zeros((16,), jnp.float32)

    def zb(i, carry):
        histv[i, :] = z16
        return carry

    lax.fori_loop(0, HR, zb, 0)
    pltpu.sync_copy(histv.at[pl.ds(sid * HRPT, HRPT)],
                    spmem.at[pl.ds(sid * HRPT, HRPT)])
    plsc.subcore_barrier()

    ones16 = jnp.ones((16,), jnp.float32)

    def hb(g, carry):
        d = dstv[g, :]
        r = lax.shift_right_logical(d, 4)
        c = lax.bitwise_and(d, 15)
        plsc.addupdate_scatter(histv, [r, c], ones16)
        return carry

    lax.fori_loop(0, EPW // 16, hb, 0)

    def mb(j, carry):
        pltpu.sync_copy(histv.at[pl.ds(j * 128, 128)],
                        spmem.at[ididv.at[j]], add=True)
        return carry

    lax.fori_loop(0, HR // 128, mb, 0)
    plsc.subcore_barrier()
    pltpu.sync_copy(spmem.at[pl.ds(sid * HRPT, HRPT)], hp_ref.at[w])


@functools.partial(
    pl.kernel,
    out_type=jax.ShapeDtypeStruct((NW, RPT, D), jnp.float32),
    scratch_types=[
        pltpu.VMEM((NPB, PB), jnp.int32),
        pltpu.VMEM((NPB, PB), jnp.int32),
        pltpu.VMEM((PB, D), jnp.float32),
        pltpu.VMEM_SHARED((N, D), jnp.float32),
        pltpu.SemaphoreType.DMA,
    ],
    mesh=_mesh(),
)
def _pool(xn_ref, src_ref, dst_ref, zrows_ref, p_ref, srcv, dstv, rbuf, acc, gsem):
    cid = lax.axis_index("c")
    sid = lax.axis_index("s")
    w = cid * NS + sid
    pltpu.sync_copy(zrows_ref, acc.at[pl.ds(sid * RPT, RPT)])
    pltpu.sync_copy(src_ref.at[w], srcv)
    pltpu.sync_copy(dst_ref.at[w], dstv)
    plsc.subcore_barrier()

    def body(j, carry):
        pltpu.async_copy(xn_ref.at[srcv.at[j]], rbuf, gsem).wait()
        pltpu.sync_copy(rbuf, acc.at[dstv.at[j]], add=True)
        return carry

    lax.fori_loop(0, NPB, body, 0)
    plsc.subcore_barrier()
    pltpu.sync_copy(acc.at[pl.ds(sid * RPT, RPT)], p_ref.at[w])


def _scale_body(x_ref, h0_ref, h1_ref, o_ref):
    inv = lax.rsqrt(h0_ref[...] + h1_ref[...])
    o_ref[...] = x_ref[...] * inv


def _out_body(p0_ref, p1_ref, h0_ref, h1_ref, w_ref, b_ref, o_ref):
    inv = lax.rsqrt(h0_ref[...] + h1_ref[...])
    pooled = (p0_ref[...] + p1_ref[...]) * inv
    acc = jnp.dot(pooled, w_ref[...], preferred_element_type=jnp.float32)
    o_ref[...] = jnp.maximum(acc + b_ref[...], 0.0)


_RB = 2000


def _scale_call(x, h0, h1):
    return pl.pallas_call(
        _scale_body,
        grid=(N // _RB,),
        in_specs=[
            pl.BlockSpec((_RB, D), lambda i: (i, 0)),
            pl.BlockSpec((_RB, 1), lambda i: (i, 0)),
            pl.BlockSpec((_RB, 1), lambda i: (i, 0)),
        ],
        out_specs=pl.BlockSpec((_RB, D), lambda i: (i, 0)),
        out_shape=jax.ShapeDtypeStruct((N, D), jnp.float32),
    )(x, h0, h1)


def _out_call(p0, p1, h0, h1, W, b2):
    return pl.pallas_call(
        _out_body,
        grid=(N // _RB,),
        in_specs=[
            pl.BlockSpec((_RB, D), lambda i: (i, 0)),
            pl.BlockSpec((_RB, D), lambda i: (i, 0)),
            pl.BlockSpec((_RB, 1), lambda i: (i, 0)),
            pl.BlockSpec((_RB, 1), lambda i: (i, 0)),
            pl.BlockSpec((D, D), lambda i: (0, 0)),
            pl.BlockSpec((1, D), lambda i: (0, 0)),
        ],
        out_specs=pl.BlockSpec((_RB, D), lambda i: (i, 0)),
        out_shape=jax.ShapeDtypeStruct((N, D), jnp.float32),
    )(p0, p1, h0, h1, W, b2)


def kernel(x, edge_index, W, b):
    src = edge_index[0].astype(jnp.int32)
    dst = edge_index[1].astype(jnp.int32)
    dst_h = dst.reshape(NW, EPW // 16, 16)
    src_p = src.reshape(NW, NPB, PB)
    dst_p = dst.reshape(NW, NPB, PB)
    idid = jnp.arange(HR, dtype=jnp.int32).reshape(HR // 128, 128)
    zrows = jnp.zeros((RPT, D), jnp.float32)

    hp = _hist(dst_h, idid)
    h = hp.reshape(NC, NS * HRPT * 16)[:, :N]
    h0 = h[0].reshape(N, 1)
    h1 = h[1].reshape(N, 1)

    xn = _scale_call(x, h0, h1)

    p = _pool(xn, src_p, dst_p, zrows)
    pr = p.reshape(NC, N, D)
    out = _out_call(pr[0], pr[1], h0, h1, W, b.reshape(1, D))
    return out


# speedup vs baseline: 3.5363x; 3.5363x over previous
import functools

import jax
import jax.numpy as jnp
from jax import lax
from jax.experimental import pallas as pl
from jax.experimental.pallas import tpu as pltpu
from jax.experimental.pallas import tpu_sc as plsc

N = 10000
E = 320000
D = 128

NC = 2
NS = 16
NW = NC * NS
EPW = E // NW

PB = 80
NPB = EPW // PB

NHALF = 5000
NPH = 5120
PRC = 4
PRL = 80


def _mesh():
    return plsc.VectorSubcoreMesh(core_axis_name="c", subcore_axis_name="s")


def _identity_rows(idref, base, rc, rcl):
    i16 = lax.iota(jnp.int32, 16)

    def ib(t, carry):
        c = t // (rcl // 16)
        k = t % (rcl // 16)
        idref[c, pl.ds(k * 16, 16)] = base + c * rcl + k * 16 + i16
        return carry

    lax.fori_loop(0, rc * (rcl // 16), ib, 0)


def _make_hist(half):
    lo = half * NHALF

    @functools.partial(
        pl.kernel,
        out_type=jax.ShapeDtypeStruct((NW * PRC, PRL, D), jnp.float32),
        scratch_types=[
            pltpu.VMEM((NPB, PB), jnp.int32),
            pltpu.VMEM((PB, D), jnp.float32),
            pltpu.VMEM((PRL, D), jnp.float32),
            pltpu.VMEM((PRC, PRL), jnp.int32),
            pltpu.VMEM_SHARED((NPH, D), jnp.float32),
        ],
        mesh=_mesh(),
    )
    def hist(dst_ref, hp_ref, dstv, onesv, stg, idr, acc):
        cid = lax.axis_index("c")
        sid = lax.axis_index("s")
        w = cid * NS + sid
        pltpu.sync_copy(dst_ref.at[w], dstv)
        z16 = jnp.zeros((16,), jnp.float32)
        ones16 = jnp.ones((16,), jnp.float32)

        def tb(t, carry):
            j = t // (PB // 16)
            k = t % (PB // 16)
            sl = pl.ds(k * 16, 16)
            d = dstv[j, sl]
            dl = d - lo
            inh = (dl >= 0) & (dl < NHALF)
            dstv[j, sl] = jnp.where(inh, dl, NHALF + (d & 63))
            return carry

        lax.fori_loop(0, NPB * (PB // 16), tb, 0)

        def ob(i, carry):
            def oc(k, carry2):
                onesv[i, pl.ds(k * 16, 16)] = ones16
                return carry2

            lax.fori_loop(0, D // 16, oc, 0)
            return carry

        lax.fori_loop(0, PB, ob, 0)

        def zb(i, carry):
            def zc(k, carry2):
                stg[i, pl.ds(k * 16, 16)] = z16
                return carry2

            lax.fori_loop(0, D // 16, zc, 0)
            return carry

        lax.fori_loop(0, PRL, zb, 0)
        _identity_rows(idr, sid * (PRC * PRL), PRC, PRL)

        def zs(c, carry):
            pltpu.sync_copy(stg, acc.at[idr.at[c]])
            return carry

        lax.fori_loop(0, PRC, zs, 0)
        plsc.subcore_barrier()

        def hb(j, carry):
            pltpu.sync_copy(onesv, acc.at[dstv.at[j]], add=True)
            return carry

        lax.fori_loop(0, NPB, hb, 0)
        plsc.subcore_barrier()

        def rb(c, carry):
            pltpu.sync_copy(acc.at[idr.at[c]], stg)
            pltpu.sync_copy(stg, hp_ref.at[w * PRC + c])
            return carry

        lax.fori_loop(0, PRC, rb, 0)

    return hist


_hist0 = _make_hist(0)
_hist1 = _make_hist(1)


def _make_pool(half):
    lo = half * NHALF

    @functools.partial(
        pl.kernel,
        out_type=jax.ShapeDtypeStruct((NW * PRC, PRL, D), jnp.float32),
        scratch_types=[
            pltpu.VMEM((NPB, PB), jnp.int32),
            pltpu.VMEM((NPB, PB), jnp.int32),
            pltpu.VMEM((PB, D), jnp.float32),
            pltpu.VMEM((PRL, D), jnp.float32),
            pltpu.VMEM((PRC, PRL), jnp.int32),
            pltpu.VMEM_SHARED((NPH, D), jnp.float32),
            pltpu.SemaphoreType.DMA,
        ],
        mesh=_mesh(),
    )
    def pool(xn_ref, src_ref, dst_ref, p_ref, srcv, dstv, rbuf, stg, idr, acc,
             gsem):
        cid = lax.axis_index("c")
        sid = lax.axis_index("s")
        w = cid * NS + sid
        pltpu.sync_copy(src_ref.at[w], srcv)
        pltpu.sync_copy(dst_ref.at[w], dstv)
        z16 = jnp.zeros((16,), jnp.float32)

        def tb(t, carry):
            j = t // (PB // 16)
            k = t % (PB // 16)
            sl = pl.ds(k * 16, 16)
            d = dstv[j, sl]
            dl = d - lo
            inh = (dl >= 0) & (dl < NHALF)
            dstv[j, sl] = jnp.where(inh, dl, NHALF + (d & 63))
            return carry

        lax.fori_loop(0, NPB * (PB // 16), tb, 0)

        def zb(i, carry):
            def zc(k, carry2):
                stg[i, pl.ds(k * 16, 16)] = z16
                return carry2

            lax.fori_loop(0, D // 16, zc, 0)
            return carry

        lax.fori_loop(0, PRL, zb, 0)
        _identity_rows(idr, sid * (PRC * PRL), PRC, PRL)

        def zs(c, carry):
            pltpu.sync_copy(stg, acc.at[idr.at[c]])
            return carry

        lax.fori_loop(0, PRC, zs, 0)
        plsc.subcore_barrier()

        def body(j, carry):
            pltpu.async_copy(xn_ref.at[srcv.at[j]], rbuf, gsem).wait()
            pltpu.sync_copy(rbuf, acc.at[dstv.at[j]], add=True)
            return carry

        lax.fori_loop(0, NPB, body, 0)
        plsc.subcore_barrier()

        def rb(c, carry):
            pltpu.sync_copy(acc.at[idr.at[c]], stg)
            pltpu.sync_copy(stg, p_ref.at[w * PRC + c])
            return carry

        lax.fori_loop(0, PRC, rb, 0)

    return pool


_pool0 = _make_pool(0)
_pool1 = _make_pool(1)


def _invd_body(h00_ref, h01_ref, h10_ref, h11_ref, o_ref):
    d0 = h00_ref[...][:NHALF, :1] + h01_ref[...][:NHALF, :1]
    d1 = h10_ref[...][:NHALF, :1] + h11_ref[...][:NHALF, :1]
    o_ref[...] = lax.rsqrt(jnp.concatenate([d0, d1], axis=0))


def _invd_call(h00, h01, h10, h11):
    return pl.pallas_call(
        _invd_body,
        out_shape=jax.ShapeDtypeStruct((N, 1), jnp.float32),
    )(h00, h01, h10, h11)


def _scale_body(x_ref, iv_ref, o_ref):
    o_ref[...] = x_ref[...] * iv_ref[...]


def _out_body(p0_ref, p1_ref, iv_ref, w_ref, b_ref, o_ref):
    pooled = (p0_ref[...] + p1_ref[...]) * iv_ref[...]
    acc = jnp.dot(pooled, w_ref[...], preferred_element_type=jnp.float32)
    o_ref[...] = jnp.maximum(acc + b_ref[...], 0.0)


_RB = 2000


def _scale_call(x, iv):
    return pl.pallas_call(
        _scale_body,
        grid=(N // _RB,),
        in_specs=[
            pl.BlockSpec((_RB, D), lambda i: (i, 0)),
            pl.BlockSpec((_RB, 1), lambda i: (i, 0)),
        ],
        out_specs=pl.BlockSpec((_RB, D), lambda i: (i, 0)),
        out_shape=jax.ShapeDtypeStruct((N, D), jnp.float32),
    )(x, iv)


def _out_call(p0, p1, iv, W, b2):
    return pl.pallas_call(
        _out_body,
        grid=(N // _RB,),
        in_specs=[
            pl.BlockSpec((_RB, D), lambda i: (i, 0)),
            pl.BlockSpec((_RB, D), lambda i: (i, 0)),
            pl.BlockSpec((_RB, 1), lambda i: (i, 0)),
            pl.BlockSpec((D, D), lambda i: (0, 0)),
            pl.BlockSpec((1, D), lambda i: (0, 0)),
        ],
        out_specs=pl.BlockSpec((_RB, D), lambda i: (i, 0)),
        out_shape=jax.ShapeDtypeStruct((N, D), jnp.float32),
    )(p0, p1, iv, W, b2)


def kernel(x, edge_index, W, b):
    src = edge_index[0].astype(jnp.int32)
    dst = edge_index[1].astype(jnp.int32)
    src_p = src.reshape(NW, NPB, PB)
    dst_p = dst.reshape(NW, NPB, PB)

    hh0 = _hist0(dst_p).reshape(NC, NPH, D)
    hh1 = _hist1(dst_p).reshape(NC, NPH, D)
    iv = _invd_call(hh0[0], hh0[1], hh1[0], hh1[1])

    xn = _scale_call(x, iv)

    ph0 = _pool0(xn, src_p, dst_p).reshape(NC, NPH, D)
    ph1 = _pool1(xn, src_p, dst_p).reshape(NC, NPH, D)
    p0 = jnp.concatenate([ph0[0, :NHALF], ph1[0, :NHALF]], axis=0)
    p1 = jnp.concatenate([ph0[1, :NHALF], ph1[1, :NHALF]], axis=0)
    out = _out_call(p0, p1, iv, W, b.reshape(1, D))
    return out


# speedup vs baseline: 4.2365x; 1.1980x over previous
import functools

import jax
import jax.numpy as jnp
from jax import lax
from jax.experimental import pallas as pl
from jax.experimental.pallas import tpu as pltpu
from jax.experimental.pallas import tpu_sc as plsc

N = 10000
E = 320000
D = 128

NC = 2
NS = 16
NW = NC * NS
EPW = E // NW

PB = 80
NPB = EPW // PB

NHALF = 5000
NPH = 5120
PRC = 4
PRL = 80


def _mesh():
    return plsc.VectorSubcoreMesh(core_axis_name="c", subcore_axis_name="s")


def _identity_rows(idref, base, rc, rcl):
    i16 = lax.iota(jnp.int32, 16)

    def ib(t, carry):
        c = t // (rcl // 16)
        k = t % (rcl // 16)
        idref[c, pl.ds(k * 16, 16)] = base + c * rcl + k * 16 + i16
        return carry

    lax.fori_loop(0, rc * (rcl // 16), ib, 0)


def _make_hist(half):
    lo = half * NHALF

    @functools.partial(
        pl.kernel,
        out_type=jax.ShapeDtypeStruct((NW * PRC, PRL, D), jnp.float32),
        scratch_types=[
            pltpu.VMEM((NPB, PB), jnp.int32),
            pltpu.VMEM((PB, D), jnp.float32),
            pltpu.VMEM((PRL, D), jnp.float32),
            pltpu.VMEM((PRC, PRL), jnp.int32),
            pltpu.VMEM_SHARED((NPH, D), jnp.float32),
            pltpu.SemaphoreType.DMA,
        ],
        mesh=_mesh(),
    )
    def hist(dst_ref, hp_ref, dstv, onesv, stg, idr, acc, ssem):
        cid = lax.axis_index("c")
        sid = lax.axis_index("s")
        w = cid * NS + sid
        pltpu.sync_copy(dst_ref.at[w], dstv)
        z16 = jnp.zeros((16,), jnp.float32)
        ones16 = jnp.ones((16,), jnp.float32)

        def tb(t, carry):
            j = t // (PB // 16)
            k = t % (PB // 16)
            sl = pl.ds(k * 16, 16)
            d = dstv[j, sl]
            dl = d - lo
            inh = (dl >= 0) & (dl < NHALF)
            dstv[j, sl] = jnp.where(inh, dl, NHALF + (d & 63))
            return carry

        lax.fori_loop(0, NPB * (PB // 16), tb, 0)

        def ob(i, carry):
            def oc(k, carry2):
                onesv[i, pl.ds(k * 16, 16)] = ones16
                return carry2

            lax.fori_loop(0, D // 16, oc, 0)
            return carry

        lax.fori_loop(0, PB, ob, 0)

        def zb(i, carry):
            def zc(k, carry2):
                stg[i, pl.ds(k * 16, 16)] = z16
                return carry2

            lax.fori_loop(0, D // 16, zc, 0)
            return carry

        lax.fori_loop(0, PRL, zb, 0)
        _identity_rows(idr, sid * (PRC * PRL), PRC, PRL)

        def zs(c, carry):
            pltpu.sync_copy(stg, acc.at[idr.at[c]])
            return carry

        lax.fori_loop(0, PRC, zs, 0)
        plsc.subcore_barrier()

        def hb(j, carry):
            pltpu.async_copy(onesv, acc.at[dstv.at[j]], ssem, add=True)
            return carry

        lax.fori_loop(0, NPB, hb, 0)

        def dr(j, carry):
            pltpu.make_async_copy(onesv, acc.at[dstv.at[j]], ssem).wait()
            return carry

        lax.fori_loop(0, NPB, dr, 0)
        plsc.subcore_barrier()

        def rb(c, carry):
            pltpu.sync_copy(acc.at[idr.at[c]], stg)
            pltpu.sync_copy(stg, hp_ref.at[w * PRC + c])
            return carry

        lax.fori_loop(0, PRC, rb, 0)

    return hist


_hist0 = _make_hist(0)
_hist1 = _make_hist(1)


def _make_pool(half):
    lo = half * NHALF

    @functools.partial(
        pl.kernel,
        out_type=jax.ShapeDtypeStruct((NW * PRC, PRL, D), jnp.float32),
        scratch_types=[
            pltpu.VMEM((NPB, PB), jnp.int32),
            pltpu.VMEM((NPB, PB), jnp.int32),
            pltpu.VMEM((PB, D), jnp.float32),
            pltpu.VMEM((PB, D), jnp.float32),
            pltpu.VMEM((PRL, D), jnp.float32),
            pltpu.VMEM((PRC, PRL), jnp.int32),
            pltpu.VMEM_SHARED((NPH, D), jnp.float32),
            pltpu.SemaphoreType.DMA,
            pltpu.SemaphoreType.DMA,
            pltpu.SemaphoreType.DMA,
        ],
        mesh=_mesh(),
    )
    def pool(xn_ref, src_ref, dst_ref, p_ref, srcv, dstv, rbuf0, rbuf1, stg,
             idr, acc, gsem, ssem0, ssem1):
        cid = lax.axis_index("c")
        sid = lax.axis_index("s")
        w = cid * NS + sid
        pltpu.sync_copy(src_ref.at[w], srcv)
        pltpu.sync_copy(dst_ref.at[w], dstv)
        z16 = jnp.zeros((16,), jnp.float32)

        def tb(t, carry):
            j = t // (PB // 16)
            k = t % (PB // 16)
            sl = pl.ds(k * 16, 16)
            d = dstv[j, sl]
            dl = d - lo
            inh = (dl >= 0) & (dl < NHALF)
            dstv[j, sl] = jnp.where(inh, dl, NHALF + (d & 63))
            return carry

        lax.fori_loop(0, NPB * (PB // 16), tb, 0)

        def zb(i, carry):
            def zc(k, carry2):
                stg[i, pl.ds(k * 16, 16)] = z16
                return carry2

            lax.fori_loop(0, D // 16, zc, 0)
            return carry

        lax.fori_loop(0, PRL, zb, 0)
        _identity_rows(idr, sid * (PRC * PRL), PRC, PRL)

        def zs(c, carry):
            pltpu.sync_copy(stg, acc.at[idr.at[c]])
            return carry

        lax.fori_loop(0, PRC, zs, 0)
        plsc.subcore_barrier()

        def step(j, rbuf, ssem):
            @pl.when(j >= 2)
            def _():
                pltpu.make_async_copy(rbuf, acc.at[idr.at[0]], ssem).wait()

            pltpu.async_copy(xn_ref.at[srcv.at[j]], rbuf, gsem).wait()
            pltpu.async_copy(rbuf, acc.at[dstv.at[j]], ssem, add=True)

        def body(j, carry):
            @pl.when((j & 1) == 0)
            def _():
                step(j, rbuf0, ssem0)

            @pl.when((j & 1) == 1)
            def _():
                step(j, rbuf1, ssem1)

            return carry

        lax.fori_loop(0, NPB, body, 0)
        pltpu.make_async_copy(rbuf0, acc.at[idr.at[0]], ssem0).wait()
        pltpu.make_async_copy(rbuf1, acc.at[idr.at[0]], ssem1).wait()
        plsc.subcore_barrier()

        def rb(c, carry):
            pltpu.sync_copy(acc.at[idr.at[c]], stg)
            pltpu.sync_copy(stg, p_ref.at[w * PRC + c])
            return carry

        lax.fori_loop(0, PRC, rb, 0)

    return pool


_pool0 = _make_pool(0)
_pool1 = _make_pool(1)


def _invd_body(h00_ref, h01_ref, h10_ref, h11_ref, o_ref):
    d0 = h00_ref[...][:NHALF, :1] + h01_ref[...][:NHALF, :1]
    d1 = h10_ref[...][:NHALF, :1] + h11_ref[...][:NHALF, :1]
    o_ref[...] = lax.rsqrt(jnp.concatenate([d0, d1], axis=0))


def _invd_call(h00, h01, h10, h11):
    return pl.pallas_call(
        _invd_body,
        out_shape=jax.ShapeDtypeStruct((N, 1), jnp.float32),
    )(h00, h01, h10, h11)


def _scale_body(x_ref, iv_ref, o_ref):
    o_ref[...] = x_ref[...] * iv_ref[...]


def _out_body(p0_ref, p1_ref, iv_ref, w_ref, b_ref, o_ref):
    pooled = (p0_ref[...] + p1_ref[...]) * iv_ref[...]
    acc = jnp.dot(pooled, w_ref[...], preferred_element_type=jnp.float32)
    o_ref[...] = jnp.maximum(acc + b_ref[...], 0.0)


_RB = 2000


def _scale_call(x, iv):
    return pl.pallas_call(
        _scale_body,
        grid=(N // _RB,),
        in_specs=[
            pl.BlockSpec((_RB, D), lambda i: (i, 0)),
            pl.BlockSpec((_RB, 1), lambda i: (i, 0)),
        ],
        out_specs=pl.BlockSpec((_RB, D), lambda i: (i, 0)),
        out_shape=jax.ShapeDtypeStruct((N, D), jnp.float32),
    )(x, iv)


def _out_call(p0, p1, iv, W, b2):
    return pl.pallas_call(
        _out_body,
        grid=(N // _RB,),
        in_specs=[
            pl.BlockSpec((_RB, D), lambda i: (i, 0)),
            pl.BlockSpec((_RB, D), lambda i: (i, 0)),
            pl.BlockSpec((_RB, 1), lambda i: (i, 0)),
            pl.BlockSpec((D, D), lambda i: (0, 0)),
            pl.BlockSpec((1, D), lambda i: (0, 0)),
        ],
        out_specs=pl.BlockSpec((_RB, D), lambda i: (i, 0)),
        out_shape=jax.ShapeDtypeStruct((N, D), jnp.float32),
    )(p0, p1, iv, W, b2)


def kernel(x, edge_index, W, b):
    src = edge_index[0].astype(jnp.int32)
    dst = edge_index[1].astype(jnp.int32)
    src_p = src.reshape(NW, NPB, PB)
    dst_p = dst.reshape(NW, NPB, PB)

    hh0 = _hist0(dst_p).reshape(NC, NPH, D)
    hh1 = _hist1(dst_p).reshape(NC, NPH, D)
    iv = _invd_call(hh0[0], hh0[1], hh1[0], hh1[1])

    xn = _scale_call(x, iv)

    ph0 = _pool0(xn, src_p, dst_p).reshape(NC, NPH, D)
    ph1 = _pool1(xn, src_p, dst_p).reshape(NC, NPH, D)
    p0 = jnp.concatenate([ph0[0, :NHALF], ph1[0, :NHALF]], axis=0)
    p1 = jnp.concatenate([ph0[1, :NHALF], ph1[1, :NHALF]], axis=0)
    out = _out_call(p0, p1, iv, W, b.reshape(1, D))
    return out
